# Initial kernel scaffold; baseline (speedup 1.0000x reference)
import functools

import jax
import jax.numpy as jnp
from jax import lax
from jax.experimental import pallas as pl
from jax.experimental.pallas import tpu as pltpu
from jax.experimental.pallas import tpu_sc as plsc

F32 = jnp.float32
NC, NS, LANES = 2, 16, 16
NW = NC * NS
CHUNK = 80
PW = 16
NGRAPH = 64
NANCH = 5


def _zero_vmem(buf, rows, cols):
    z16 = jnp.zeros((LANES,), F32)

    def zr(r, _):
        def zc(c, __):
            buf[r, pl.ds(c * LANES, LANES)] = z16
            return 0

        return lax.fori_loop(0, cols // LANES, zc, 0)

    lax.fori_loop(0, rows, zr, 0)


def _make_segsum(N, Dp, E):
    EW = E // NW
    NCH = EW // CHUNK
    RPT = N // NS
    ZR = 125
    mesh = plsc.VectorSubcoreMesh(core_axis_name="c", subcore_axis_name="s")

    @functools.partial(
        pl.kernel,
        out_type=jax.ShapeDtypeStruct((NC * N, Dp), F32),
        mesh=mesh,
        scratch_types=[
            pltpu.VMEM((CHUNK,), jnp.int32),
            pltpu.VMEM((CHUNK,), jnp.int32),
            pltpu.VMEM((CHUNK, Dp), F32),
            pltpu.VMEM((ZR, Dp), F32),
            pltpu.VMEM_SHARED((N, Dp), F32),
            pltpu.SemaphoreType.DMA,
        ],
    )
    def seg_kernel(table, src2d, dst2d, out, idx_s, idx_d, rows, zbuf, acc, sem):
        c = lax.axis_index("c")
        s = lax.axis_index("s")
        wid = c * NS + s
        _zero_vmem(zbuf, ZR, Dp)

        def zcopy(q, _):
            pltpu.sync_copy(zbuf, acc.at[pl.ds(s * RPT + q * ZR, ZR)])
            return 0

        lax.fori_loop(0, RPT // ZR, zcopy, 0)
        plsc.subcore_barrier()

        base = wid * NCH

        def step(j, _):
            pltpu.sync_copy(src2d.at[base + j], idx_s)
            pltpu.sync_copy(dst2d.at[base + j], idx_d)
            pltpu.async_copy(table.at[idx_s], rows, sem).wait()
            pltpu.sync_copy(rows, acc.at[idx_d], add=True)
            return 0

        lax.fori_loop(0, NCH, step, 0)
        plsc.subcore_barrier()
        pltpu.sync_copy(
            acc.at[pl.ds(s * RPT, RPT)], out.at[pl.ds(c * N + s * RPT, RPT)]
        )

    return seg_kernel


def _make_funcedge(N, Dp, E):
    EW = E // NW
    NCH = EW // CHUNK
    RPT = N // NS
    ZR = 125
    mesh = plsc.VectorSubcoreMesh(core_axis_name="c", subcore_axis_name="s")

    @functools.partial(
        pl.kernel,
        out_type=(
            jax.ShapeDtypeStruct((NC * N, Dp), F32),
            jax.ShapeDtypeStruct((NC * N, PW), F32),
        ),
        mesh=mesh,
        scratch_types=[
            pltpu.VMEM((CHUNK,), jnp.int32),
            pltpu.VMEM((CHUNK,), jnp.int32),
            pltpu.VMEM((CHUNK, Dp), F32),
            pltpu.VMEM((CHUNK, PW), F32),
            pltpu.VMEM((CHUNK, PW), F32),
            pltpu.VMEM((CHUNK, PW), F32),
            pltpu.VMEM((ZR, Dp), F32),
            pltpu.VMEM((ZR, PW), F32),
            pltpu.VMEM_SHARED((N, Dp), F32),
            pltpu.VMEM_SHARED((N, PW), F32),
            pltpu.SemaphoreType.DMA,
        ],
    )
    def func_kernel(table, pmat, src2d, dst2d, out_h, out_w,
                    idx_s, idx_d, rows, ps, pd, wbuf, zbuf, zwb, acc_h, acc_w,
                    sem):
        c = lax.axis_index("c")
        s = lax.axis_index("s")
        wid = c * NS + s
        _zero_vmem(zbuf, ZR, Dp)
        _zero_vmem(zwb, ZR, PW)
        _zero_vmem(wbuf, CHUNK, PW)

        def zcopy(q, _):
            pltpu.sync_copy(zbuf, acc_h.at[pl.ds(s * RPT + q * ZR, ZR)])
            pltpu.sync_copy(zwb, acc_w.at[pl.ds(s * RPT + q * ZR, ZR)])
            return 0

        lax.fori_loop(0, RPT // ZR, zcopy, 0)
        plsc.subcore_barrier()

        base = wid * NCH
        iota16 = lax.iota(jnp.int32, LANES)

        def step(j, _):
            pltpu.sync_copy(src2d.at[base + j], idx_s)
            pltpu.sync_copy(dst2d.at[base + j], idx_d)
            d1 = pltpu.async_copy(table.at[idx_s], rows, sem)
            d2 = pltpu.async_copy(pmat.at[idx_s], ps, sem)
            d3 = pltpu.async_copy(pmat.at[idx_d], pd, sem)
            d1.wait()
            d2.wait()
            d3.wait()
            for g in range(CHUNK // LANES):
                rowi = g * LANES + iota16
                logit = []
                for a in range(NANCH):
                    ca = jnp.full((LANES,), a, jnp.int32)
                    la = plsc.load_gather(ps, [rowi, ca]) + plsc.load_gather(
                        pd, [rowi, ca]
                    )
                    logit.append(la)
                m = logit[0]
                for a in range(1, NANCH):
                    m = jnp.maximum(m, logit[a])
                ex = [jnp.exp(la - m) for la in logit]
                tot = ex[0]
                for a in range(1, NANCH):
                    tot = tot + ex[a]
                inv = 1.0 / tot
                for a in range(NANCH):
                    ca = jnp.full((LANES,), a, jnp.int32)
                    plsc.store_scatter(wbuf, [rowi, ca], ex[a] * inv)
            pltpu.sync_copy(rows, acc_h.at[idx_d], add=True)
            pltpu.sync_copy(wbuf, acc_w.at[idx_d], add=True)
            return 0

        lax.fori_loop(0, NCH, step, 0)
        plsc.subcore_barrier()
        pltpu.sync_copy(
            acc_h.at[pl.ds(s * RPT, RPT)], out_h.at[pl.ds(c * N + s * RPT, RPT)]
        )
        pltpu.sync_copy(
            acc_w.at[pl.ds(s * RPT, RPT)], out_w.at[pl.ds(c * N + s * RPT, RPT)]
        )

    return func_kernel


def _dot(a, b):
    return jnp.dot(a, b, preferred_element_type=F32)


def _bn_rows(y, g, b):
    m = jnp.mean(y, axis=0, keepdims=True)
    d = y - m
    v = jnp.mean(d * d, axis=0, keepdims=True)
    return g * d * lax.rsqrt(v + 1e-5) + b


def _masked_bn_rows(y, g, b, kept, k):
    m = jnp.sum(y * kept, axis=0, keepdims=True) / k
    d = y - m
    v = jnp.sum(d * d * kept, axis=0, keepdims=True) / k
    return g * d * lax.rsqrt(v + 1e-5) + b


def _tc_call(body, out_shapes):
    return pl.pallas_call(body, out_shape=out_shapes)


def _tc_pre_body(x, ancT, anc, fp0T, roi, xs_o, xf_o, p0_o):
    xv = x[...]
    logits = _dot(xv, ancT[...])
    col = lax.broadcasted_iota(jnp.int32, (1, 8), 1)
    logits = jnp.where(col < NANCH, logits, -1e30)
    e = jnp.exp(logits - jnp.max(logits, axis=1, keepdims=True))
    w = e / jnp.sum(e, axis=1, keepdims=True)
    xs_o[...] = xv + _dot(w, anc[...])
    xf = xv * roi[...]
    xf_o[...] = xf
    p0_o[...] = _dot(xf, fp0T[...])


def _tc_struct1_body(N, k, xs, parts, W1, b1, W2, b2, g, bb, patt, pattT,
                     h1_o, kept_o):
    hin = xs[...] + parts[pl.ds(0, N), :] + parts[pl.ds(N, N), :]
    h = jnp.maximum(_dot(hin, W1[...]) + b1[...], 0.0)
    h = _dot(h, W2[...]) + b2[...]
    h = _bn_rows(h, g[...], bb[...])
    h = jnp.maximum(h, 0.0)
    score_c = _dot(h, patt[...])
    score_r = lax.dot_general(pattT[...], h, (((1,), (1,)), ((), ())),
                              preferred_element_type=F32)
    idx_r = lax.broadcasted_iota(jnp.int32, (1, N), 1)
    RB = 80

    def step(j, _):
        sb = lax.dynamic_slice(score_c, (j * RB, 0), (RB, 1))
        ib = j * RB + lax.broadcasted_iota(jnp.int32, (RB, 1), 0)
        gt = (score_r > sb).astype(F32)
        eq = jnp.logical_and(score_r == sb, idx_r < ib).astype(F32)
        rank = jnp.sum(gt + eq, axis=1, keepdims=True)
        kept_o[pl.ds(j * RB, RB), :] = (rank < k).astype(F32)
        return 0

    lax.fori_loop(0, N // RB, step, 0)
    h1_o[...] = kept_o[...] * h * jnp.tanh(score_c)


def _tc_post_body(N, k, do_relu, Hi, parts, W1, b1, W2, b2, g, bb, kept, h_o):
    hin = Hi[...] + parts[pl.ds(0, N), :] + parts[pl.ds(N, N), :]
    y = jnp.maximum(_dot(hin, W1[...]) + b1[...], 0.0)
    y = _dot(y, W2[...]) + b2[...]
    kv = kept[...]
    y = _masked_bn_rows(y, g[...], bb[...], kv, k)
    if do_relu:
        y = jnp.maximum(y, 0.0)
    h_o[...] = kv * y


def _tc_struct3_body(N, k, B, Hi, parts, W1, b1, W2, b2, g, bb, kept,
                     batch_col, z_o):
    hin = Hi[...] + parts[pl.ds(0, N), :] + parts[pl.ds(N, N), :]
    y = jnp.maximum(_dot(hin, W1[...]) + b1[...], 0.0)
    y = _dot(y, W2[...]) + b2[...]
    kv = kept[...]
    y = _masked_bn_rows(y, g[...], bb[...], kv, k)
    yk = kv * y
    bc = batch_col[...]
    iota_b = lax.broadcasted_iota(jnp.int32, (1, NGRAPH), 1)
    ohT = jnp.logical_and(bc == iota_b, kv > 0).astype(F32)
    cnt = lax.dot_general(ohT, jnp.ones((N, 1), F32), (((0,), (0,)), ((), ())),
                          preferred_element_type=F32)
    num = lax.dot_general(ohT, yk, (((0,), (0,)), ((), ())),
                          preferred_element_type=F32)
    z_o[:, pl.ds(0, 64)] = num / jnp.maximum(cnt, 1.0)
    neg = jnp.float32(-3e38)

    def step(gi, _):
        mask = jnp.logical_and(bc == gi, kv > 0)
        val = jnp.where(mask, y, neg)
        mx = jnp.max(val, axis=0, keepdims=True)
        mx = jnp.where(mx <= neg * 0.5, 0.0, mx)
        z_o[pl.ds(gi, 1), pl.ds(64, 64)] = mx
        return 0

    lax.fori_loop(0, B, step, 0)


def _tc_func_body(N, do_relu, hf, hparts, wparts, Apad, W1, b1, W2, b2, g, bb,
                  AnT, hf_o, p_o):
    wagg = wparts[pl.ds(0, N), :] + wparts[pl.ds(N, N), :]
    agg = (hparts[pl.ds(0, N), :] + hparts[pl.ds(N, N), :]
           + _dot(wagg, Apad[...]))
    hin = hf[...] + agg
    y = jnp.maximum(_dot(hin, W1[...]) + b1[...], 0.0)
    y = _dot(y, W2[...]) + b2[...]
    y = _bn_rows(y, g[...], bb[...])
    if do_relu:
        y = jnp.maximum(y, 0.0)
    hf_o[...] = y
    p_o[...] = _dot(y, AnT[...])


def _tc_func3_body(N, B, hf, hparts, wparts, Apad, W1, b1, W2, b2, g, bb,
                   batch_col, roi, z_o):
    wagg = wparts[pl.ds(0, N), :] + wparts[pl.ds(N, N), :]
    agg = (hparts[pl.ds(0, N), :] + hparts[pl.ds(N, N), :]
           + _dot(wagg, Apad[...]))
    hin = hf[...] + agg
    y = jnp.maximum(_dot(hin, W1[...]) + b1[...], 0.0)
    y = _dot(y, W2[...]) + b2[...]
    y = _bn_rows(y, g[...], bb[...])
    bc = jnp.where(roi[...] > 0, batch_col[...], 0)
    iota_b = lax.broadcasted_iota(jnp.int32, (1, NGRAPH), 1)
    ohT = (bc == iota_b).astype(F32)
    cnt = lax.dot_general(ohT, jnp.ones((N, 1), F32), (((0,), (0,)), ((), ())),
                          preferred_element_type=F32)
    num = lax.dot_general(ohT, y, (((0,), (0,)), ((), ())),
                          preferred_element_type=F32)
    z_o[:, pl.ds(0, 64)] = num / jnp.maximum(cnt, 1.0)
    neg = jnp.float32(-3e38)

    def step(gi, _):
        val = jnp.where(bc == gi, y, neg)
        mx = jnp.max(val, axis=0, keepdims=True)
        mx = jnp.where(mx <= neg * 0.5, 0.0, mx)
        z_o[pl.ds(gi, 1), pl.ds(64, 64)] = mx
        return 0

    lax.fori_loop(0, B, step, 0)


def _tc_fuse_body(zs, zf, c1W, c1b, c1g, c1bb, c2W, c2b, c2g, c2bb, c3W, c3b,
                  logits_o):
    z = jnp.concatenate([zs[...], zf[...]], axis=1)
    z = jnp.maximum(
        _bn_rows(_dot(z, c1W[...]) + c1b[...], c1g[...], c1bb[...]), 0.0)
    z = jnp.maximum(
        _bn_rows(_dot(z, c2W[...]) + c2b[...], c2g[...], c2bb[...]), 0.0)
    logits_o[...] = _dot(z, c3W[...]) + c3b[...]


def _pad_anchors(A, rows):
    a, d = A.shape
    return jnp.zeros((rows, d), F32).at[:a].set(A)


def kernel(x, edge_attr_struct, params, batch, edge_index_struct,
           edge_index_func, roi_mask):
    N, D = x.shape
    E = edge_index_struct.shape[1]
    H = 64
    B = NGRAPH
    k = int(N * 0.5)

    batch_col = batch.astype(jnp.int32).reshape(N, 1)
    roi_col = roi_mask.astype(F32).reshape(N, 1)

    anc = _pad_anchors(params['sp_anchors'], 8)
    ancT = anc.T
    fp = [_pad_anchors(a, PW) for a in params['fp_anchors']]
    fpT = [a.T for a in fp]

    src_s = edge_index_struct[0].reshape(E // CHUNK, CHUNK)
    dst_s = edge_index_struct[1].reshape(E // CHUNK, CHUNK)
    src_f = edge_index_func[0].reshape(E // CHUNK, CHUNK)
    dst_f = edge_index_func[1].reshape(E // CHUNK, CHUNK)

    sds = jax.ShapeDtypeStruct

    xs, xf, p0 = _tc_call(
        _tc_pre_body,
        (sds((N, D), F32), sds((N, D), F32), sds((N, PW), F32)),
    )(x, ancT, anc, fpT[0], roi_col)

    seg128 = _make_segsum(N, D, E)
    seg64 = _make_segsum(N, H, E)
    fedge128 = _make_funcedge(N, D, E)
    fedge64 = _make_funcedge(N, H, E)

    sparts = seg128(xs, src_s, dst_s)
    pre = params['pre']
    h1, kept = _tc_call(
        functools.partial(_tc_struct1_body, N, k),
        (sds((N, H), F32), sds((N, 1), F32)),
    )(xs, sparts, pre['W1'], pre['b1'].reshape(1, H), pre['W2'],
      pre['b2'].reshape(1, H), params['pre_g'].reshape(1, H),
      params['pre_b'].reshape(1, H), params['pool_att'].reshape(H, 1),
      params['pool_att'].reshape(1, H))

    p0_ = params['post'][0]
    parts = seg64(h1, src_s, dst_s)
    h2 = _tc_call(
        functools.partial(_tc_post_body, N, k, True),
        sds((N, H), F32),
    )(h1, parts, p0_['W1'], p0_['b1'].reshape(1, H), p0_['W2'],
      p0_['b2'].reshape(1, H), p0_['g'].reshape(1, H), p0_['b'].reshape(1, H),
      kept)

    p1_ = params['post'][1]
    parts = seg64(h2, src_s, dst_s)
    z_struct = _tc_call(
        functools.partial(_tc_struct3_body, N, k, B),
        sds((B, 2 * H), F32),
    )(h2, parts, p1_['W1'], p1_['b1'].reshape(1, H), p1_['W2'],
      p1_['b2'].reshape(1, H), p1_['g'].reshape(1, H), p1_['b'].reshape(1, H),
      kept, batch_col)

    hf = xf
    pmat = p0
    z_func = None
    for l in range(3):
        fe = fedge128 if l == 0 else fedge64
        hparts, wparts = fe(hf, pmat, src_f, dst_f)
        fl = params['func'][l]
        if l < 2:
            hf, pmat = _tc_call(
                functools.partial(_tc_func_body, N, True),
                (sds((N, H), F32), sds((N, PW), F32)),
            )(hf, hparts, wparts, fp[l], fl['W1'], fl['b1'].reshape(1, H),
              fl['W2'], fl['b2'].reshape(1, H), fl['g'].reshape(1, H),
              fl['b'].reshape(1, H), fpT[l + 1])
        else:
            z_func = _tc_call(
                functools.partial(_tc_func3_body, N, B),
                sds((B, 2 * H), F32),
            )(hf, hparts, wparts, fp[l], fl['W1'], fl['b1'].reshape(1, H),
              fl['W2'], fl['b2'].reshape(1, H), fl['g'].reshape(1, H),
              fl['b'].reshape(1, H), batch_col, roi_col)

    logits = _tc_call(
        _tc_fuse_body,
        sds((B, 2), F32),
    )(z_struct, z_func, params['c1_W'], params['c1_b'].reshape(1, 2 * H),
      params['c1_g'].reshape(1, 2 * H), params['c1_bb'].reshape(1, 2 * H),
      params['c2_W'], params['c2_b'].reshape(1, H),
      params['c2_g'].reshape(1, H), params['c2_bb'].reshape(1, H),
      params['c3_W'], params['c3_b'].reshape(1, 2))

    return logits, z_struct, z_func


# speedup vs baseline: 9.7958x; 9.7958x over previous
import functools

import jax
import jax.numpy as jnp
from jax import lax
from jax.experimental import pallas as pl
from jax.experimental.pallas import tpu as pltpu
from jax.experimental.pallas import tpu_sc as plsc

F32 = jnp.float32
NC, NS, LANES = 2, 16, 16
NW = NC * NS
CHUNK = 80
PW = 16
PP = 8
NGRAPH = 64
NANCH = 5


RPT = 632
NPAD = NS * RPT


def _fill_zeros(buf, rows, cols):
    z16 = jnp.zeros((LANES,), F32)

    def zr(r, _):
        def zc(c, __):
            buf[r, pl.ds(c * LANES, LANES)] = z16
            return 0

        return lax.fori_loop(0, cols // LANES, zc, 0)

    lax.fori_loop(0, rows, zr, 0)


def _fill_zeros_1d(buf, n):
    z16 = jnp.zeros((LANES,), F32)

    def zc(c, _):
        buf[pl.ds(c * LANES, LANES)] = z16
        return 0

    lax.fori_loop(0, n // LANES, zc, 0)


def _zero_acc(zbuf, acc, base, zsem):
    nz = RPT // 8
    for q in range(nz):
        pltpu.async_copy(zbuf, acc.at[pl.ds(base + q * 8, 8)], zsem)
    for q in range(nz):
        pltpu.make_async_copy(zbuf, acc.at[pl.ds(base, 8)], zsem).wait()


def _make_segsum(N, Dp, E):
    EW = E // NW
    NCH = EW // CHUNK
    mesh = plsc.VectorSubcoreMesh(core_axis_name="c", subcore_axis_name="s")

    @functools.partial(
        pl.kernel,
        out_type=jax.ShapeDtypeStruct((NC * NPAD, Dp), F32),
        mesh=mesh,
        compiler_params=pltpu.CompilerParams(needs_layout_passes=False, use_tc_tiling_on_sc=False),
        scratch_types=[
            pltpu.VMEM((NCH, CHUNK), jnp.int32),
            pltpu.VMEM((NCH, CHUNK), jnp.int32),
            pltpu.VMEM((CHUNK, Dp), F32),
            pltpu.VMEM((8, Dp), F32),
            pltpu.VMEM_SHARED((NPAD, Dp), F32),
            pltpu.SemaphoreType.DMA,
            pltpu.SemaphoreType.DMA,
        ],
    )
    def seg_kernel(table, src3d, dst3d, out, idx_s, idx_d, rows, zbuf, acc,
                   sem, zsem):
        c = lax.axis_index("c")
        s = lax.axis_index("s")
        wid = c * NS + s
        pltpu.sync_copy(src3d.at[wid], idx_s)
        pltpu.sync_copy(dst3d.at[wid], idx_d)
        _fill_zeros(zbuf, 8, Dp)
        _zero_acc(zbuf, acc, s * RPT, zsem)
        plsc.subcore_barrier()

        def step(j, _):
            pltpu.async_copy(table.at[idx_s.at[j]], rows, sem).wait()
            pltpu.sync_copy(rows, acc.at[idx_d.at[j]], add=True)
            return 0

        lax.fori_loop(0, NCH, step, 0)
        plsc.subcore_barrier()
        pltpu.sync_copy(
            acc.at[pl.ds(s * RPT, RPT)], out.at[pl.ds(c * NPAD + s * RPT, RPT)]
        )

    return seg_kernel


def _make_wsum(N, E):
    EW = E // NW
    NCH = EW // CHUNK
    mesh = plsc.VectorSubcoreMesh(core_axis_name="c", subcore_axis_name="s")

    @functools.partial(
        pl.kernel,
        out_type=jax.ShapeDtypeStruct((NC * NPAD, PW), F32),
        mesh=mesh,
        compiler_params=pltpu.CompilerParams(needs_layout_passes=False, use_tc_tiling_on_sc=False),
        scratch_types=[
            pltpu.VMEM((NCH, CHUNK), jnp.int32),
            pltpu.VMEM((NCH, CHUNK), jnp.int32),
            pltpu.VMEM((N * PP,), F32),
            pltpu.VMEM((CHUNK * PW,), F32),
            pltpu.VMEM((CHUNK, PW), F32),
            pltpu.VMEM((8, PW), F32),
            pltpu.VMEM_SHARED((NPAD, PW), F32),
            pltpu.SemaphoreType.DMA,
        ],
    )
    def wsum_kernel(pmat, src3d, dst3d, out_w,
                    idx_s, idx_d, pvm, wbuf, wrow, zwb, acc_w, zsem):
        c = lax.axis_index("c")
        s = lax.axis_index("s")
        wid = c * NS + s
        pltpu.sync_copy(src3d.at[wid], idx_s)
        pltpu.sync_copy(dst3d.at[wid], idx_d)
        pltpu.sync_copy(pmat, pvm)
        _fill_zeros(zwb, 8, PW)
        _fill_zeros_1d(wbuf, CHUNK * PW)
        _zero_acc(zwb, acc_w, s * RPT, zsem)
        plsc.subcore_barrier()

        iota16 = lax.iota(jnp.int32, LANES)

        def step(j, _):
            for g in range(CHUNK // LANES):
                vs = idx_s[j, pl.ds(g * LANES, LANES)] * PP
                vd = idx_d[j, pl.ds(g * LANES, LANES)] * PP
                logit = []
                for a in range(NANCH):
                    la = plsc.load_gather(pvm, [vs + a]) + plsc.load_gather(
                        pvm, [vd + a]
                    )
                    logit.append(la)
                m = logit[0]
                for a in range(1, NANCH):
                    m = jnp.maximum(m, logit[a])
                ex = [jnp.exp(la - m) for la in logit]
                tot = ex[0]
                for a in range(1, NANCH):
                    tot = tot + ex[a]
                inv = 1.0 / tot
                flat = (g * LANES + iota16) * PW
                for a in range(NANCH):
                    plsc.store_scatter(wbuf, [flat + a], ex[a] * inv)
            for r in range(CHUNK):
                wrow[r, :] = wbuf[pl.ds(r * PW, PW)]
            pltpu.sync_copy(wrow, acc_w.at[idx_d.at[j]], add=True)
            return 0

        lax.fori_loop(0, NCH, step, 0)
        plsc.subcore_barrier()
        pltpu.sync_copy(
            acc_w.at[pl.ds(s * RPT, RPT)],
            out_w.at[pl.ds(c * NPAD + s * RPT, RPT)],
        )

    return wsum_kernel


def _dot(a, b):
    return jnp.dot(a, b, preferred_element_type=F32)


def _bn_rows(y, g, b):
    m = jnp.mean(y, axis=0, keepdims=True)
    d = y - m
    v = jnp.mean(d * d, axis=0, keepdims=True)
    return g * d * lax.rsqrt(v + 1e-5) + b


def _masked_bn_rows(y, g, b, kept, k):
    m = jnp.sum(y * kept, axis=0, keepdims=True) / k
    d = y - m
    v = jnp.sum(d * d * kept, axis=0, keepdims=True) / k
    return g * d * lax.rsqrt(v + 1e-5) + b


def _tc_call(body, out_shapes):
    return pl.pallas_call(body, out_shape=out_shapes)


def _tc_pre_body(x, ancT, anc, fp0T, roi, xs_o, xf_o, p0_o):
    xv = x[...]
    logits = _dot(xv, ancT[...])
    col = lax.broadcasted_iota(jnp.int32, (1, 8), 1)
    logits = jnp.where(col < NANCH, logits, -1e30)
    e = jnp.exp(logits - jnp.max(logits, axis=1, keepdims=True))
    w = e / jnp.sum(e, axis=1, keepdims=True)
    xs_o[...] = xv + _dot(w, anc[...])
    xf = xv * roi[...]
    xf_o[...] = xf
    p0_o[...] = _dot(xf, fp0T[...])


def _tc_struct1a_body(N, xs, partsA, partsB, W1, b1, W2, b2, g, bb, patt,
                      h_o, sc_o):
    agg = jnp.concatenate(
        [partsA[pl.ds(0, N), :] + partsA[pl.ds(NPAD, N), :],
         partsB[pl.ds(0, N), :] + partsB[pl.ds(NPAD, N), :]], axis=1)
    hin = xs[...] + agg
    h = jnp.maximum(_dot(hin, W1[...]) + b1[...], 0.0)
    h = _dot(h, W2[...]) + b2[...]
    h = _bn_rows(h, g[...], bb[...])
    h = jnp.maximum(h, 0.0)
    h_o[...] = h
    sc_o[...] = _dot(h, patt[...])


def _tc_struct1b_body(N, k, h, score_c, score_r, h1_o, kept_o):
    sr = score_r[...]
    idx_r = lax.broadcasted_iota(jnp.int32, (1, N), 1)
    RB = 80

    def step(j, _):
        sb = score_c[pl.ds(j * RB, RB), :]
        ib = j * RB + lax.broadcasted_iota(jnp.int32, (RB, 1), 0)
        gt = (sr > sb).astype(F32)
        eq = jnp.logical_and(sr == sb, idx_r < ib).astype(F32)
        rank = jnp.sum(gt + eq, axis=1, keepdims=True)
        kept_o[pl.ds(j * RB, RB), :] = (rank < k).astype(F32)
        return 0

    lax.fori_loop(0, N // RB, step, 0)
    h1_o[...] = kept_o[...] * h[...] * jnp.tanh(score_c[...])


def _tc_post_body(N, k, do_relu, Hi, parts, W1, b1, W2, b2, g, bb, kept, h_o):
    hin = Hi[...] + parts[pl.ds(0, N), :] + parts[pl.ds(NPAD, N), :]
    y = jnp.maximum(_dot(hin, W1[...]) + b1[...], 0.0)
    y = _dot(y, W2[...]) + b2[...]
    kv = kept[...]
    y = _masked_bn_rows(y, g[...], bb[...], kv, k)
    if do_relu:
        y = jnp.maximum(y, 0.0)
    h_o[...] = kv * y


def _tc_struct3_body(N, k, B, Hi, parts, W1, b1, W2, b2, g, bb, kept,
                     batch_col, z_o):
    hin = Hi[...] + parts[pl.ds(0, N), :] + parts[pl.ds(NPAD, N), :]
    y = jnp.maximum(_dot(hin, W1[...]) + b1[...], 0.0)
    y = _dot(y, W2[...]) + b2[...]
    kv = kept[...]
    y = _masked_bn_rows(y, g[...], bb[...], kv, k)
    yk = kv * y
    bc = batch_col[...]
    iota_b = lax.broadcasted_iota(jnp.int32, (1, NGRAPH), 1)
    ohT = jnp.logical_and(bc == iota_b, kv > 0).astype(F32)
    cnt = lax.dot_general(ohT, jnp.ones((N, 1), F32), (((0,), (0,)), ((), ())),
                          preferred_element_type=F32)
    num = lax.dot_general(ohT, yk, (((0,), (0,)), ((), ())),
                          preferred_element_type=F32)
    z_o[:, pl.ds(0, 64)] = num / jnp.maximum(cnt, 1.0)
    neg = jnp.float32(-3e38)

    def step(gi, _):
        mask = jnp.logical_and(bc == gi, kv > 0)
        val = jnp.where(mask, y, neg)
        mx = jnp.max(val, axis=0, keepdims=True)
        mx = jnp.where(mx <= neg * 0.5, 0.0, mx)
        z_o[pl.ds(gi, 1), pl.ds(64, 64)] = mx
        return 0

    lax.fori_loop(0, B, step, 0)


def _tc_func_body(N, do_relu, hf, hparts, wparts, Apad, W1, b1, W2, b2, g, bb,
                  AnT, hf_o, p_o):
    wagg = wparts[pl.ds(0, N), :] + wparts[pl.ds(NPAD, N), :]
    agg = (hparts[pl.ds(0, N), :] + hparts[pl.ds(NPAD, N), :]
           + _dot(wagg, Apad[...]))
    hin = hf[...] + agg
    y = jnp.maximum(_dot(hin, W1[...]) + b1[...], 0.0)
    y = _dot(y, W2[...]) + b2[...]
    y = _bn_rows(y, g[...], bb[...])
    if do_relu:
        y = jnp.maximum(y, 0.0)
    hf_o[...] = y
    p_o[...] = _dot(y, AnT[...])


def _tc_func0_body(N, hf, hpartsA, hpartsB, wparts, Apad, W1, b1, W2, b2, g,
                   bb, AnT, hf_o, p_o):
    wagg = wparts[pl.ds(0, N), :] + wparts[pl.ds(NPAD, N), :]
    agg = jnp.concatenate(
        [hpartsA[pl.ds(0, N), :] + hpartsA[pl.ds(NPAD, N), :],
         hpartsB[pl.ds(0, N), :] + hpartsB[pl.ds(NPAD, N), :]], axis=1)
    agg = agg + _dot(wagg, Apad[...])
    hin = hf[...] + agg
    y = jnp.maximum(_dot(hin, W1[...]) + b1[...], 0.0)
    y = _dot(y, W2[...]) + b2[...]
    y = _bn_rows(y, g[...], bb[...])
    y = jnp.maximum(y, 0.0)
    hf_o[...] = y
    p_o[...] = _dot(y, AnT[...])


def _tc_func3_body(N, B, hf, hparts, wparts, Apad, W1, b1, W2, b2, g, bb,
                   batch_col, roi, z_o):
    wagg = wparts[pl.ds(0, N), :] + wparts[pl.ds(NPAD, N), :]
    agg = (hparts[pl.ds(0, N), :] + hparts[pl.ds(NPAD, N), :]
           + _dot(wagg, Apad[...]))
    hin = hf[...] + agg
    y = jnp.maximum(_dot(hin, W1[...]) + b1[...], 0.0)
    y = _dot(y, W2[...]) + b2[...]
    y = _bn_rows(y, g[...], bb[...])
    bc = jnp.where(roi[...] > 0, batch_col[...], 0)
    iota_b = lax.broadcasted_iota(jnp.int32, (1, NGRAPH), 1)
    ohT = (bc == iota_b).astype(F32)
    cnt = lax.dot_general(ohT, jnp.ones((N, 1), F32), (((0,), (0,)), ((), ())),
                          preferred_element_type=F32)
    num = lax.dot_general(ohT, y, (((0,), (0,)), ((), ())),
                          preferred_element_type=F32)
    z_o[:, pl.ds(0, 64)] = num / jnp.maximum(cnt, 1.0)
    neg = jnp.float32(-3e38)

    def step(gi, _):
        val = jnp.where(bc == gi, y, neg)
        mx = jnp.max(val, axis=0, keepdims=True)
        mx = jnp.where(mx <= neg * 0.5, 0.0, mx)
        z_o[pl.ds(gi, 1), pl.ds(64, 64)] = mx
        return 0

    lax.fori_loop(0, B, step, 0)


def _tc_fuse_body(zs, zf, c1W, c1b, c1g, c1bb, c2W, c2b, c2g, c2bb, c3W, c3b,
                  logits_o):
    z = jnp.concatenate([zs[...], zf[...]], axis=1)
    z = jnp.maximum(
        _bn_rows(_dot(z, c1W[...]) + c1b[...], c1g[...], c1bb[...]), 0.0)
    z = jnp.maximum(
        _bn_rows(_dot(z, c2W[...]) + c2b[...], c2g[...], c2bb[...]), 0.0)
    logits_o[...] = _dot(z, c3W[...]) + c3b[...]


def _pad_anchors(A, rows):
    a, d = A.shape
    return jnp.zeros((rows, d), F32).at[:a].set(A)


def kernel(x, edge_attr_struct, params, batch, edge_index_struct,
           edge_index_func, roi_mask):
    N, D = x.shape
    E = edge_index_struct.shape[1]
    H = 64
    B = NGRAPH
    k = int(N * 0.5)

    batch_col = batch.astype(jnp.int32).reshape(N, 1)
    roi_col = roi_mask.astype(F32).reshape(N, 1)

    anc = _pad_anchors(params['sp_anchors'], 8)
    ancT = anc.T
    fp = [_pad_anchors(a, PW) for a in params['fp_anchors']]
    fp8T = [_pad_anchors(a, PP).T for a in params['fp_anchors']]

    nch = E // NW // CHUNK
    src_s = edge_index_struct[0].reshape(NW, nch, CHUNK)
    dst_s = edge_index_struct[1].reshape(NW, nch, CHUNK)
    src_f = edge_index_func[0].reshape(NW, nch, CHUNK)
    dst_f = edge_index_func[1].reshape(NW, nch, CHUNK)

    sds = jax.ShapeDtypeStruct

    xs, xf, p0 = _tc_call(
        _tc_pre_body,
        (sds((N, D), F32), sds((N, D), F32), sds((N, PP), F32)),
    )(x, ancT, anc, fp8T[0], roi_col)

    seg64 = _make_segsum(N, H, E)
    wsum = _make_wsum(N, E)

    spartsA = seg64(xs[:, :H], src_s, dst_s)
    spartsB = seg64(xs[:, H:], src_s, dst_s)
    pre = params['pre']
    hmid, score = _tc_call(
        functools.partial(_tc_struct1a_body, N),
        (sds((N, H), F32), sds((N, 1), F32)),
    )(xs, spartsA, spartsB, pre['W1'], pre['b1'].reshape(1, H), pre['W2'],
      pre['b2'].reshape(1, H), params['pre_g'].reshape(1, H),
      params['pre_b'].reshape(1, H), params['pool_att'].reshape(H, 1))
    h1, kept = _tc_call(
        functools.partial(_tc_struct1b_body, N, k),
        (sds((N, H), F32), sds((N, 1), F32)),
    )(hmid, score, score.reshape(1, N))

    p0_ = params['post'][0]
    parts = seg64(h1, src_s, dst_s)
    h2 = _tc_call(
        functools.partial(_tc_post_body, N, k, True),
        sds((N, H), F32),
    )(h1, parts, p0_['W1'], p0_['b1'].reshape(1, H), p0_['W2'],
      p0_['b2'].reshape(1, H), p0_['g'].reshape(1, H), p0_['b'].reshape(1, H),
      kept)

    p1_ = params['post'][1]
    parts = seg64(h2, src_s, dst_s)
    z_struct = _tc_call(
        functools.partial(_tc_struct3_body, N, k, B),
        sds((B, 2 * H), F32),
    )(h2, parts, p1_['W1'], p1_['b1'].reshape(1, H), p1_['W2'],
      p1_['b2'].reshape(1, H), p1_['g'].reshape(1, H), p1_['b'].reshape(1, H),
      kept, batch_col)

    hf = xf
    pmat = p0
    z_func = None
    for l in range(3):
        fl = params['func'][l]
        if l == 0:
            hpartsA = seg64(hf[:, :H], src_f, dst_f)
            hpartsB = seg64(hf[:, H:], src_f, dst_f)
            wparts = wsum(pmat.reshape(-1), src_f, dst_f)
            hf, pmat = _tc_call(
                functools.partial(_tc_func0_body, N),
                (sds((N, H), F32), sds((N, PP), F32)),
            )(hf, hpartsA, hpartsB, wparts, fp[l], fl['W1'],
              fl['b1'].reshape(1, H), fl['W2'], fl['b2'].reshape(1, H),
              fl['g'].reshape(1, H), fl['b'].reshape(1, H), fp8T[l + 1])
            continue
        hparts = seg64(hf, src_f, dst_f)
        wparts = wsum(pmat.reshape(-1), src_f, dst_f)
        if l < 2:
            hf, pmat = _tc_call(
                functools.partial(_tc_func_body, N, True),
                (sds((N, H), F32), sds((N, PP), F32)),
            )(hf, hparts, wparts, fp[l], fl['W1'], fl['b1'].reshape(1, H),
              fl['W2'], fl['b2'].reshape(1, H), fl['g'].reshape(1, H),
              fl['b'].reshape(1, H), fp8T[l + 1])
        else:
            z_func = _tc_call(
                functools.partial(_tc_func3_body, N, B),
                sds((B, 2 * H), F32),
            )(hf, hparts, wparts, fp[l], fl['W1'], fl['b1'].reshape(1, H),
              fl['W2'], fl['b2'].reshape(1, H), fl['g'].reshape(1, H),
              fl['b'].reshape(1, H), batch_col, roi_col)

    logits = _tc_call(
        _tc_fuse_body,
        sds((B, 2), F32),
    )(z_struct, z_func, params['c1_W'], params['c1_b'].reshape(1, 2 * H),
      params['c1_g'].reshape(1, 2 * H), params['c1_bb'].reshape(1, 2 * H),
      params['c2_W'], params['c2_b'].reshape(1, H),
      params['c2_g'].reshape(1, H), params['c2_bb'].reshape(1, H),
      params['c3_W'], params['c3_b'].reshape(1, 2))

    return logits, z_struct, z_func
